# baseline (speedup 1.0000x reference)
import functools

import jax
import jax.numpy as jnp
from jax import lax
from jax.experimental import pallas as pl
from jax.experimental.pallas import tpu as pltpu
from jax.experimental.pallas import tpu_sc as plsc

N = 10000
E = 320000
D1 = 128
DH = 64
D2 = 64
NC, NS = 2, 16
NW = NC * NS
K = 128
CH = 80
CHT = NC * CH
E_PAD = NW * CH * K
N_PAD = 10240
RT = N_PAD // NS
DW = 8
NBUF = 5
LOOK = 3

_sc_mesh = plsc.VectorSubcoreMesh(
    core_axis_name="c", subcore_axis_name="s", num_cores=NC, num_subcores=NS)
_sc_params = pltpu.CompilerParams(use_tc_tiling_on_sc=False)


@functools.partial(
    pl.kernel,
    out_type=jax.ShapeDtypeStruct((NC * N_PAD, DW), jnp.float32),
    mesh=_sc_mesh,
    scratch_types=[
        pltpu.VMEM((CH, K), jnp.int32),
        pltpu.VMEM((K + RT, DW), jnp.float32),
        pltpu.SemaphoreType.DMA,
        pltpu.VMEM_SHARED((N_PAD, DW), jnp.float32),
    ],
    compiler_params=_sc_params,
)
def _deg_kernel(dst_hbm, const_hbm, out_hbm, dstv, cv, dsem, acc):
    c = lax.axis_index("c")
    s = lax.axis_index("s")
    wid = c * NS + s
    pltpu.sync_copy(dst_hbm.at[wid], dstv)
    pltpu.sync_copy(const_hbm, cv)
    pltpu.sync_copy(cv.at[pl.ds(K, RT)], acc.at[pl.ds(s * RT, RT)])
    plsc.subcore_barrier()
    ones = cv.at[pl.ds(0, K)]
    FD = 8

    def grp(g, carry):
        base = g * FD
        for j in range(FD):
            pltpu.async_copy(ones, acc.at[dstv.at[base + j]], dsem, add=True)
        for j in range(FD):
            pltpu.make_async_copy(ones, acc.at[dstv.at[base + j]], dsem).wait()
        return carry

    lax.fori_loop(0, CH // FD, grp, 0)
    plsc.subcore_barrier()
    pltpu.sync_copy(acc.at[pl.ds(s * RT, RT)],
                    out_hbm.at[pl.ds(c * N_PAD + s * RT, RT)])


def _make_agg(D, split_cols):
    chunks = CHT if split_cols else CH

    @functools.partial(
        pl.kernel,
        out_type=jax.ShapeDtypeStruct((NC * N_PAD, D), jnp.float32),
        mesh=_sc_mesh,
        scratch_types=[
            pltpu.VMEM((chunks, K), jnp.int32),
            pltpu.VMEM((chunks, K), jnp.int32),
            [pltpu.VMEM((K, D), jnp.float32) for _ in range(NBUF)],
            [pltpu.SemaphoreType.DMA for _ in range(NBUF)],
            [pltpu.SemaphoreType.DMA for _ in range(NBUF)],
            pltpu.VMEM_SHARED((N_PAD, D), jnp.float32),
        ],
        compiler_params=_sc_params,
    )
    def _agg(g_hbm, src_hbm, dst_hbm, out_hbm, srcv, dstv, bufs, gsems, ssems,
             acc):
        c = lax.axis_index("c")
        s = lax.axis_index("s")
        if split_cols:
            gref = g_hbm.at[c]
            erow = s
        else:
            gref = g_hbm
            erow = c * NS + s
        pltpu.sync_copy(gref.at[pl.ds(s * RT, RT)], acc.at[pl.ds(s * RT, RT)])
        pltpu.sync_copy(src_hbm.at[erow], srcv)
        pltpu.sync_copy(dst_hbm.at[erow], dstv)
        plsc.subcore_barrier()
        for j in range(LOOK):
            pltpu.async_copy(gref.at[srcv.at[j]], bufs[j], gsems[j])

        def grp(g, carry):
            base = g * NBUF
            for b in range(NBUF):
                ch = base + b
                pltpu.make_async_copy(
                    gref.at[srcv.at[ch]], bufs[b], gsems[b]).wait()
                pltpu.async_copy(bufs[b], acc.at[dstv.at[ch]], ssems[b],
                                 add=True)
                f = ch + LOOK
                bf = (b + LOOK) % NBUF

                @pl.when(f < chunks)
                def _issue(f=f, bf=bf):
                    @pl.when(f >= NBUF)
                    def _drain():
                        pltpu.make_async_copy(
                            bufs[bf], acc.at[dstv.at[f - NBUF]],
                            ssems[bf]).wait()

                    pltpu.async_copy(gref.at[srcv.at[f]], bufs[bf], gsems[bf])
            return carry

        lax.fori_loop(0, chunks // NBUF, grp, 0)
        for b in range(NBUF):
            ch = chunks - NBUF + b
            pltpu.make_async_copy(bufs[b], acc.at[dstv.at[ch]],
                                  ssems[b]).wait()
        plsc.subcore_barrier()
        pltpu.sync_copy(acc.at[pl.ds(s * RT, RT)],
                        out_hbm.at[pl.ds(c * N_PAD + s * RT, RT)])

    return _agg


_agg1 = _make_agg(DH, split_cols=True)
_agg2 = _make_agg(D2, split_cols=False)

RB = 1024
GR = N_PAD // RB


def _dinv_col(hist_ref):
    deg = hist_ref[0] + hist_ref[1] + 1.0
    return lax.rsqrt(deg)[:, 0:1]


def _mm1_body(hist_ref, x_ref, w_ref, o_ref):
    h = jnp.dot(x_ref[...], w_ref[...], precision=lax.Precision.HIGHEST,
                preferred_element_type=jnp.float32) * _dinv_col(hist_ref)
    o_ref[0] = h[:, :DH]
    o_ref[1] = h[:, DH:]


def _mid_body(hist_ref, s_ref, b1_ref, w2_ref, o_ref):
    dcol = _dinv_col(hist_ref)
    t = jnp.concatenate([s_ref[0], s_ref[1]], axis=1) * dcol + b1_ref[...]
    a = jnp.maximum(t, 0.0)
    o_ref[...] = jnp.dot(a, w2_ref[...], precision=lax.Precision.HIGHEST,
                         preferred_element_type=jnp.float32) * dcol


def _fin_body(hist_ref, s_ref, g_ref, b2_ref, o_ref):
    dcol = _dinv_col(hist_ref)
    o_ref[...] = (s_ref[0] + s_ref[1] - g_ref[...]) * dcol + b2_ref[...]


def _row_spec(d):
    return pl.BlockSpec((2, RB, d), lambda i: (0, i, 0))


def _full_spec(shape):
    return pl.BlockSpec(shape, lambda i: tuple(0 for _ in shape))


def kernel(x, edge_index, W1, b1, W2, b2):
    ei = edge_index.astype(jnp.int32)
    pad = E_PAD - E
    src = jnp.concatenate([ei[0], jnp.zeros((pad,), jnp.int32)])
    dst = jnp.concatenate([ei[1], jnp.full((pad,), N, jnp.int32)])
    src16, dst16 = src.reshape(NS, CHT, K), dst.reshape(NS, CHT, K)
    src32, dst32 = src.reshape(NW, CH, K), dst.reshape(NW, CH, K)
    xp = jnp.concatenate([x, jnp.zeros((N_PAD - N, D1), x.dtype)])
    const = jnp.concatenate([jnp.ones((K, DW), jnp.float32),
                             jnp.zeros((RT, DW), jnp.float32)])

    hist = _deg_kernel(dst32, const).reshape(NC, N_PAD, DW)

    g1 = pl.pallas_call(
        _mm1_body,
        grid=(GR,),
        out_shape=jax.ShapeDtypeStruct((NC, N_PAD, DH), jnp.float32),
        in_specs=[_row_spec(DW),
                  pl.BlockSpec((RB, D1), lambda i: (i, 0)),
                  _full_spec((D1, D1))],
        out_specs=_row_spec(DH),
    )(hist, xp, W1)

    s1 = _agg1(g1, src16, dst16).reshape(NC, N_PAD, DH)

    g2 = pl.pallas_call(
        _mid_body,
        grid=(GR,),
        out_shape=jax.ShapeDtypeStruct((N_PAD, D2), jnp.float32),
        in_specs=[_row_spec(DW),
                  _row_spec(DH),
                  _full_spec((1, D1)),
                  _full_spec((D1, D2))],
        out_specs=pl.BlockSpec((RB, D2), lambda i: (i, 0)),
    )(hist, s1, b1.reshape(1, D1), W2)

    s2 = _agg2(g2, src32, dst32).reshape(NC, N_PAD, D2)

    out = pl.pallas_call(
        _fin_body,
        grid=(GR,),
        out_shape=jax.ShapeDtypeStruct((N_PAD, D2), jnp.float32),
        in_specs=[_row_spec(DW),
                  _row_spec(D2),
                  pl.BlockSpec((RB, D2), lambda i: (i, 0)),
                  _full_spec((1, D2))],
        out_specs=pl.BlockSpec((RB, D2), lambda i: (i, 0)),
    )(hist, s2, g2, b2.reshape(1, D2))

    return out[:N]


# speedup vs baseline: 1.2497x; 1.2497x over previous
import functools

import jax
import jax.numpy as jnp
from jax import lax
from jax.experimental import pallas as pl
from jax.experimental.pallas import tpu as pltpu
from jax.experimental.pallas import tpu_sc as plsc

N = 10000
E = 320000
D1 = 128
DH = 64
D2 = 64
NC, NS = 2, 16
NW = NC * NS
K = 128
CH = 80
CHT = NC * CH
E_PAD = NW * CH * K
N_PAD = 10240
RT = N_PAD // NS
DW = 8
NBUF = 5
LOOK = 3

_sc_mesh = plsc.VectorSubcoreMesh(
    core_axis_name="c", subcore_axis_name="s", num_cores=NC, num_subcores=NS)
_sc_params = pltpu.CompilerParams(use_tc_tiling_on_sc=False)


@functools.partial(
    pl.kernel,
    out_type=jax.ShapeDtypeStruct((NC * N_PAD, DW), jnp.float32),
    mesh=_sc_mesh,
    scratch_types=[
        pltpu.VMEM((CH, K), jnp.int32),
        pltpu.VMEM((K + RT, DW), jnp.float32),
        pltpu.SemaphoreType.DMA,
        pltpu.VMEM_SHARED((N_PAD, DW), jnp.float32),
    ],
    compiler_params=_sc_params,
)
def _deg_kernel(dst_hbm, const_hbm, out_hbm, dstv, cv, dsem, acc):
    c = lax.axis_index("c")
    s = lax.axis_index("s")
    wid = c * NS + s
    pltpu.sync_copy(dst_hbm.at[wid], dstv)
    pltpu.sync_copy(const_hbm, cv)
    pltpu.sync_copy(cv.at[pl.ds(K, RT)], acc.at[pl.ds(s * RT, RT)])
    plsc.subcore_barrier()
    ones = cv.at[pl.ds(0, K)]
    FD = 8

    def grp(g, carry):
        base = g * FD
        for j in range(FD):
            pltpu.async_copy(ones, acc.at[dstv.at[base + j]], dsem, add=True)
        for j in range(FD):
            pltpu.make_async_copy(ones, acc.at[dstv.at[base + j]], dsem).wait()
        return carry

    lax.fori_loop(0, CH // FD, grp, 0)
    plsc.subcore_barrier()
    pltpu.sync_copy(acc.at[pl.ds(s * RT, RT)],
                    out_hbm.at[pl.ds(c * N_PAD + s * RT, RT)])


def _make_agg(D, split_cols, dtype=jnp.float32):
    chunks = CHT if split_cols else CH

    @functools.partial(
        pl.kernel,
        out_type=jax.ShapeDtypeStruct((NC * N_PAD, D), dtype),
        mesh=_sc_mesh,
        scratch_types=[
            pltpu.VMEM((chunks, K), jnp.int32),
            pltpu.VMEM((chunks, K), jnp.int32),
            [pltpu.VMEM((K, D), dtype) for _ in range(NBUF)],
            [pltpu.SemaphoreType.DMA for _ in range(NBUF)],
            [pltpu.SemaphoreType.DMA for _ in range(NBUF)],
            pltpu.VMEM_SHARED((N_PAD, D), dtype),
        ],
        compiler_params=_sc_params,
    )
    def _agg(g_hbm, src_hbm, dst_hbm, out_hbm, srcv, dstv, bufs, gsems, ssems,
             acc):
        c = lax.axis_index("c")
        s = lax.axis_index("s")
        if split_cols:
            gref = g_hbm.at[c]
            erow = s
        else:
            gref = g_hbm
            erow = c * NS + s
        pltpu.sync_copy(gref.at[pl.ds(s * RT, RT)], acc.at[pl.ds(s * RT, RT)])
        pltpu.sync_copy(src_hbm.at[erow], srcv)
        pltpu.sync_copy(dst_hbm.at[erow], dstv)
        plsc.subcore_barrier()
        for j in range(LOOK):
            pltpu.async_copy(gref.at[srcv.at[j]], bufs[j], gsems[j])

        def grp(g, carry):
            base = g * NBUF
            for b in range(NBUF):
                ch = base + b
                pltpu.make_async_copy(
                    gref.at[srcv.at[ch]], bufs[b], gsems[b]).wait()
                pltpu.async_copy(bufs[b], acc.at[dstv.at[ch]], ssems[b],
                                 add=True)
                f = ch + LOOK
                bf = (b + LOOK) % NBUF

                @pl.when(f < chunks)
                def _issue(f=f, bf=bf):
                    @pl.when(f >= NBUF)
                    def _drain():
                        pltpu.make_async_copy(
                            bufs[bf], acc.at[dstv.at[f - NBUF]],
                            ssems[bf]).wait()

                    pltpu.async_copy(gref.at[srcv.at[f]], bufs[bf], gsems[bf])
            return carry

        lax.fori_loop(0, chunks // NBUF, grp, 0)
        for b in range(NBUF):
            ch = chunks - NBUF + b
            pltpu.make_async_copy(bufs[b], acc.at[dstv.at[ch]],
                                  ssems[b]).wait()
        plsc.subcore_barrier()
        pltpu.sync_copy(acc.at[pl.ds(s * RT, RT)],
                        out_hbm.at[pl.ds(c * N_PAD + s * RT, RT)])

    return _agg


_agg1 = _make_agg(DH, split_cols=True, dtype=jnp.bfloat16)
_agg2 = _make_agg(D2, split_cols=False)

RB = 1024
GR = N_PAD // RB


def _dinv_col(hist_ref):
    deg = hist_ref[0] + hist_ref[1] + 1.0
    return lax.rsqrt(deg)[:, 0:1]


def _mm1_body(hist_ref, x_ref, w_ref, o_ref):
    h = (jnp.dot(x_ref[...], w_ref[...], precision=lax.Precision.HIGHEST,
                 preferred_element_type=jnp.float32)
         * _dinv_col(hist_ref)).astype(jnp.bfloat16)
    o_ref[0] = h[:, :DH]
    o_ref[1] = h[:, DH:]


def _mid_body(hist_ref, s_ref, b1_ref, w2_ref, o_ref):
    dcol = _dinv_col(hist_ref)
    t = (jnp.concatenate([s_ref[0], s_ref[1]], axis=1).astype(jnp.float32)
         * dcol + b1_ref[...])
    a = jnp.maximum(t, 0.0)
    o_ref[...] = jnp.dot(a, w2_ref[...], precision=lax.Precision.HIGHEST,
                         preferred_element_type=jnp.float32) * dcol


def _fin_body(hist_ref, s_ref, g_ref, b2_ref, o_ref):
    dcol = _dinv_col(hist_ref)
    o_ref[...] = (s_ref[0] + s_ref[1] - g_ref[...]) * dcol + b2_ref[...]


def _row_spec(d):
    return pl.BlockSpec((2, RB, d), lambda i: (0, i, 0))


def _full_spec(shape):
    return pl.BlockSpec(shape, lambda i: tuple(0 for _ in shape))


def kernel(x, edge_index, W1, b1, W2, b2):
    ei = edge_index.astype(jnp.int32)
    pad = E_PAD - E
    src = jnp.concatenate([ei[0], jnp.zeros((pad,), jnp.int32)])
    dst = jnp.concatenate([ei[1], jnp.full((pad,), N, jnp.int32)])
    src16, dst16 = src.reshape(NS, CHT, K), dst.reshape(NS, CHT, K)
    src32, dst32 = src.reshape(NW, CH, K), dst.reshape(NW, CH, K)
    xp = jnp.concatenate([x, jnp.zeros((N_PAD - N, D1), x.dtype)])
    const = jnp.concatenate([jnp.ones((K, DW), jnp.float32),
                             jnp.zeros((RT, DW), jnp.float32)])

    hist = _deg_kernel(dst32, const).reshape(NC, N_PAD, DW)

    g1 = pl.pallas_call(
        _mm1_body,
        grid=(GR,),
        out_shape=jax.ShapeDtypeStruct((NC, N_PAD, DH), jnp.bfloat16),
        in_specs=[_row_spec(DW),
                  pl.BlockSpec((RB, D1), lambda i: (i, 0)),
                  _full_spec((D1, D1))],
        out_specs=_row_spec(DH),
    )(hist, xp, W1)

    s1 = _agg1(g1, src16, dst16).reshape(NC, N_PAD, DH)

    g2 = pl.pallas_call(
        _mid_body,
        grid=(GR,),
        out_shape=jax.ShapeDtypeStruct((N_PAD, D2), jnp.float32),
        in_specs=[_row_spec(DW),
                  _row_spec(DH),
                  _full_spec((1, D1)),
                  _full_spec((D1, D2))],
        out_specs=pl.BlockSpec((RB, D2), lambda i: (i, 0)),
    )(hist, s1, b1.reshape(1, D1), W2)

    s2 = _agg2(g2, src32, dst32).reshape(NC, N_PAD, D2)

    out = pl.pallas_call(
        _fin_body,
        grid=(GR,),
        out_shape=jax.ShapeDtypeStruct((N_PAD, D2), jnp.float32),
        in_specs=[_row_spec(DW),
                  _row_spec(D2),
                  pl.BlockSpec((RB, D2), lambda i: (i, 0)),
                  _full_spec((1, D2))],
        out_specs=pl.BlockSpec((RB, D2), lambda i: (i, 0)),
    )(hist, s2, g2, b2.reshape(1, D2))

    return out[:N]


# speedup vs baseline: 1.6048x; 1.2841x over previous
import functools

import jax
import jax.numpy as jnp
from jax import lax
from jax.experimental import pallas as pl
from jax.experimental.pallas import tpu as pltpu
from jax.experimental.pallas import tpu_sc as plsc

N = 10000
E = 320000
D1 = 128
DH = 64
D2 = 64
NC, NS = 2, 16
NW = NC * NS
K = 128
CH = 80
CHT = NC * CH
E_PAD = NW * CH * K
N_PAD = 10240
RT = N_PAD // NS
DW = 8
NBUF = 5
LOOK = 3

_sc_mesh = plsc.VectorSubcoreMesh(
    core_axis_name="c", subcore_axis_name="s", num_cores=NC, num_subcores=NS)
_sc_params = pltpu.CompilerParams(use_tc_tiling_on_sc=False)


@functools.partial(
    pl.kernel,
    out_type=jax.ShapeDtypeStruct((NC * N_PAD, DW), jnp.float32),
    mesh=_sc_mesh,
    scratch_types=[
        pltpu.VMEM((CH, K), jnp.int32),
        pltpu.VMEM((K + RT, DW), jnp.float32),
        pltpu.SemaphoreType.DMA,
        pltpu.VMEM_SHARED((N_PAD, DW), jnp.float32),
    ],
    compiler_params=_sc_params,
)
def _deg_kernel(dst_hbm, const_hbm, out_hbm, dstv, cv, dsem, acc):
    c = lax.axis_index("c")
    s = lax.axis_index("s")
    wid = c * NS + s
    pltpu.sync_copy(dst_hbm.at[wid], dstv)
    pltpu.sync_copy(const_hbm, cv)
    pltpu.sync_copy(cv.at[pl.ds(K, RT)], acc.at[pl.ds(s * RT, RT)])
    plsc.subcore_barrier()
    ones = cv.at[pl.ds(0, K)]
    FD = 8

    def grp(g, carry):
        base = g * FD
        for j in range(FD):
            pltpu.async_copy(ones, acc.at[dstv.at[base + j]], dsem, add=True)
        for j in range(FD):
            pltpu.make_async_copy(ones, acc.at[dstv.at[base + j]], dsem).wait()
        return carry

    lax.fori_loop(0, CH // FD, grp, 0)
    plsc.subcore_barrier()
    pltpu.sync_copy(acc.at[pl.ds(s * RT, RT)],
                    out_hbm.at[pl.ds(c * N_PAD + s * RT, RT)])


def _make_agg(D, split_cols, dtype=jnp.float32):
    chunks = CHT if split_cols else CH

    @functools.partial(
        pl.kernel,
        out_type=jax.ShapeDtypeStruct((NC * N_PAD, D), dtype),
        mesh=_sc_mesh,
        scratch_types=[
            pltpu.VMEM((chunks, K), jnp.int32),
            pltpu.VMEM((chunks, K), jnp.int32),
            [pltpu.VMEM((K, D), dtype) for _ in range(NBUF)],
            [pltpu.SemaphoreType.DMA for _ in range(NBUF)],
            [pltpu.SemaphoreType.DMA for _ in range(NBUF)],
            pltpu.VMEM_SHARED((N_PAD, D), dtype),
        ],
        compiler_params=_sc_params,
    )
    def _agg(g_hbm, src_hbm, dst_hbm, out_hbm, srcv, dstv, bufs, gsems, ssems,
             acc):
        c = lax.axis_index("c")
        s = lax.axis_index("s")
        if split_cols:
            gref = g_hbm.at[c]
            erow = s
        else:
            gref = g_hbm
            erow = c * NS + s
        pltpu.sync_copy(gref.at[pl.ds(s * RT, RT)], acc.at[pl.ds(s * RT, RT)])
        pltpu.sync_copy(src_hbm.at[erow], srcv)
        pltpu.sync_copy(dst_hbm.at[erow], dstv)
        plsc.subcore_barrier()
        for j in range(LOOK):
            pltpu.async_copy(gref.at[srcv.at[j]], bufs[j], gsems[j])

        def grp(g, carry):
            base = g * NBUF
            for b in range(NBUF):
                ch = base + b
                pltpu.make_async_copy(
                    gref.at[srcv.at[ch]], bufs[b], gsems[b]).wait()
                pltpu.async_copy(bufs[b], acc.at[dstv.at[ch]], ssems[b],
                                 add=True)
                f = ch + LOOK
                bf = (b + LOOK) % NBUF

                @pl.when(f < chunks)
                def _issue(f=f, bf=bf):
                    @pl.when(f >= NBUF)
                    def _drain():
                        pltpu.make_async_copy(
                            bufs[bf], acc.at[dstv.at[f - NBUF]],
                            ssems[bf]).wait()

                    pltpu.async_copy(gref.at[srcv.at[f]], bufs[bf], gsems[bf])
            return carry

        lax.fori_loop(0, chunks // NBUF, grp, 0)
        for b in range(NBUF):
            ch = chunks - NBUF + b
            pltpu.make_async_copy(bufs[b], acc.at[dstv.at[ch]],
                                  ssems[b]).wait()
        plsc.subcore_barrier()
        pltpu.sync_copy(acc.at[pl.ds(s * RT, RT)],
                        out_hbm.at[pl.ds(c * N_PAD + s * RT, RT)])

    return _agg


_agg1 = _make_agg(DH, split_cols=True, dtype=jnp.bfloat16)
_agg2 = _make_agg(D2, split_cols=False, dtype=jnp.bfloat16)

RB = 1024
GR = N_PAD // RB


def _dinv_col(hist_ref):
    deg = hist_ref[0] + hist_ref[1] + 1.0
    return lax.rsqrt(deg)[:, 0:1]


def _mm1_body(hist_ref, x_ref, w_ref, o_ref):
    h = (jnp.dot(x_ref[...], w_ref[...], precision=lax.Precision.HIGHEST,
                 preferred_element_type=jnp.float32)
         * _dinv_col(hist_ref)).astype(jnp.bfloat16)
    o_ref[0] = h[:, :DH]
    o_ref[1] = h[:, DH:]


def _mid_body(hist_ref, s_ref, b1_ref, w2_ref, o_ref):
    dcol = _dinv_col(hist_ref)
    t = (jnp.concatenate([s_ref[0], s_ref[1]], axis=1).astype(jnp.float32)
         * dcol + b1_ref[...])
    a = jnp.maximum(t, 0.0)
    o_ref[...] = (jnp.dot(a, w2_ref[...], precision=lax.Precision.HIGHEST,
                          preferred_element_type=jnp.float32)
                  * dcol).astype(jnp.bfloat16)


def _fin_body(hist_ref, s_ref, g_ref, b2_ref, o_ref):
    dcol = _dinv_col(hist_ref)
    s = (s_ref[0].astype(jnp.float32) + s_ref[1].astype(jnp.float32)
         - g_ref[...].astype(jnp.float32))
    o_ref[...] = s * dcol + b2_ref[...]


def _row_spec(d):
    return pl.BlockSpec((2, RB, d), lambda i: (0, i, 0))


def _full_spec(shape):
    return pl.BlockSpec(shape, lambda i: tuple(0 for _ in shape))


def kernel(x, edge_index, W1, b1, W2, b2):
    ei = edge_index.astype(jnp.int32)
    pad = E_PAD - E
    src = jnp.concatenate([ei[0], jnp.zeros((pad,), jnp.int32)])
    dst = jnp.concatenate([ei[1], jnp.full((pad,), N, jnp.int32)])
    src16, dst16 = src.reshape(NS, CHT, K), dst.reshape(NS, CHT, K)
    src32, dst32 = src.reshape(NW, CH, K), dst.reshape(NW, CH, K)
    xp = jnp.concatenate([x, jnp.zeros((N_PAD - N, D1), x.dtype)])
    const = jnp.concatenate([jnp.ones((K, DW), jnp.float32),
                             jnp.zeros((RT, DW), jnp.float32)])

    hist = _deg_kernel(dst32, const).reshape(NC, N_PAD, DW)

    g1 = pl.pallas_call(
        _mm1_body,
        grid=(GR,),
        out_shape=jax.ShapeDtypeStruct((NC, N_PAD, DH), jnp.bfloat16),
        in_specs=[_row_spec(DW),
                  pl.BlockSpec((RB, D1), lambda i: (i, 0)),
                  _full_spec((D1, D1))],
        out_specs=_row_spec(DH),
    )(hist, xp, W1)

    s1 = _agg1(g1, src16, dst16).reshape(NC, N_PAD, DH)

    g2 = pl.pallas_call(
        _mid_body,
        grid=(GR,),
        out_shape=jax.ShapeDtypeStruct((N_PAD, D2), jnp.bfloat16),
        in_specs=[_row_spec(DW),
                  _row_spec(DH),
                  _full_spec((1, D1)),
                  _full_spec((D1, D2))],
        out_specs=pl.BlockSpec((RB, D2), lambda i: (i, 0)),
    )(hist, s1, b1.reshape(1, D1), W2)

    s2 = _agg2(g2, src32, dst32).reshape(NC, N_PAD, D2)

    out = pl.pallas_call(
        _fin_body,
        grid=(GR,),
        out_shape=jax.ShapeDtypeStruct((N_PAD, D2), jnp.float32),
        in_specs=[_row_spec(DW),
                  _row_spec(D2),
                  pl.BlockSpec((RB, D2), lambda i: (i, 0)),
                  _full_spec((1, D2))],
        out_specs=pl.BlockSpec((RB, D2), lambda i: (i, 0)),
    )(hist, s2, g2, b2.reshape(1, D2))

    return out[:N]


# speedup vs baseline: 1.6307x; 1.0161x over previous
import functools

import jax
import jax.numpy as jnp
from jax import lax
from jax.experimental import pallas as pl
from jax.experimental.pallas import tpu as pltpu
from jax.experimental.pallas import tpu_sc as plsc

N = 10000
E = 320000
D1 = 128
DH = 64
D2 = 64
NC, NS = 2, 16
NW = NC * NS
K = 128
CH = 80
CHT = NC * CH
E_PAD = NW * CH * K
N_PAD = 10240
RT = N_PAD // NS
DW = 8
NBUF = 5
LOOK = 3

_sc_mesh = plsc.VectorSubcoreMesh(
    core_axis_name="c", subcore_axis_name="s", num_cores=NC, num_subcores=NS)
_sc_params = pltpu.CompilerParams(use_tc_tiling_on_sc=False)


@functools.partial(
    pl.kernel,
    out_type=jax.ShapeDtypeStruct((NC * N_PAD, DW), jnp.float32),
    mesh=_sc_mesh,
    scratch_types=[
        pltpu.VMEM((CH, K), jnp.int32),
        pltpu.VMEM((K + RT, DW), jnp.float32),
        pltpu.SemaphoreType.DMA,
        pltpu.VMEM_SHARED((N_PAD, DW), jnp.float32),
    ],
    compiler_params=_sc_params,
)
def _deg_kernel(dst_hbm, const_hbm, out_hbm, dstv, cv, dsem, acc):
    c = lax.axis_index("c")
    s = lax.axis_index("s")
    wid = c * NS + s
    pltpu.sync_copy(dst_hbm.at[wid], dstv)
    pltpu.sync_copy(const_hbm, cv)
    pltpu.sync_copy(cv.at[pl.ds(K, RT)], acc.at[pl.ds(s * RT, RT)])
    plsc.subcore_barrier()
    ones = cv.at[pl.ds(0, K)]
    FD = 8

    def grp(g, carry):
        base = g * FD
        for j in range(FD):
            pltpu.async_copy(ones, acc.at[dstv.at[base + j]], dsem, add=True)
        for j in range(FD):
            pltpu.make_async_copy(ones, acc.at[dstv.at[base + j]], dsem).wait()
        return carry

    lax.fori_loop(0, CH // FD, grp, 0)
    plsc.subcore_barrier()
    pltpu.sync_copy(acc.at[pl.ds(s * RT, RT)],
                    out_hbm.at[pl.ds(c * N_PAD + s * RT, RT)])


def _make_agg(D, split_cols, dtype=jnp.float32):
    chunks = CHT if split_cols else CH

    @functools.partial(
        pl.kernel,
        out_type=jax.ShapeDtypeStruct((NC * N_PAD, D), dtype),
        mesh=_sc_mesh,
        scratch_types=[
            pltpu.VMEM((chunks, K), jnp.int32),
            pltpu.VMEM((chunks, K), jnp.int32),
            [pltpu.VMEM((K, D), dtype) for _ in range(NBUF)],
            [pltpu.SemaphoreType.DMA for _ in range(NBUF)],
            [pltpu.SemaphoreType.DMA for _ in range(NBUF)],
            pltpu.VMEM_SHARED((N_PAD, D), dtype),
        ],
        compiler_params=_sc_params,
    )
    def _agg(g_hbm, src_hbm, dst_hbm, out_hbm, srcv, dstv, bufs, gsems, ssems,
             acc):
        c = lax.axis_index("c")
        s = lax.axis_index("s")
        if split_cols:
            gref = g_hbm.at[c]
            erow = s
        else:
            gref = g_hbm
            erow = c * NS + s
        rows = pl.ds(s * RT, RT)
        pltpu.sync_copy(gref.at[rows], acc.at[rows])
        pltpu.sync_copy(src_hbm.at[erow], srcv)
        pltpu.sync_copy(dst_hbm.at[erow], dstv)
        plsc.subcore_barrier()
        for j in range(LOOK):
            pltpu.async_copy(gref.at[srcv.at[j]], bufs[j], gsems[j])

        def grp(g, carry):
            base = g * NBUF
            for b in range(NBUF):
                ch = base + b
                pltpu.make_async_copy(
                    gref.at[srcv.at[ch]], bufs[b], gsems[b]).wait()
                pltpu.async_copy(bufs[b], acc.at[dstv.at[ch]], ssems[b],
                                 add=True)
                f = ch + LOOK
                bf = (b + LOOK) % NBUF

                @pl.when(f < chunks)
                def _issue(f=f, bf=bf):
                    @pl.when(f >= NBUF)
                    def _drain():
                        pltpu.make_async_copy(
                            bufs[bf], acc.at[dstv.at[f - NBUF]],
                            ssems[bf]).wait()

                    pltpu.async_copy(gref.at[srcv.at[f]], bufs[bf], gsems[bf])
            return carry

        lax.fori_loop(0, chunks // NBUF, grp, 0)
        for b in range(NBUF):
            ch = chunks - NBUF + b
            pltpu.make_async_copy(bufs[b], acc.at[dstv.at[ch]],
                                  ssems[b]).wait()
        plsc.subcore_barrier()
        pltpu.sync_copy(acc.at[pl.ds(s * RT, RT)],
                        out_hbm.at[pl.ds(c * N_PAD + s * RT, RT)])

    return _agg


_agg1 = _make_agg(DH, split_cols=True, dtype=jnp.bfloat16)
_agg2 = _make_agg(D2, split_cols=False, dtype=jnp.bfloat16)

RB = 1024
GR = N_PAD // RB


def _dinv_col(hist_ref):
    deg = hist_ref[0] + hist_ref[1] + 1.0
    return lax.rsqrt(deg)[:, 0:1]


def _mm1_body(hist_ref, x_ref, w_ref, o_ref):
    h = (jnp.dot(x_ref[...], w_ref[...], precision=lax.Precision.HIGHEST,
                 preferred_element_type=jnp.float32)
         * _dinv_col(hist_ref)).astype(jnp.bfloat16)
    o_ref[0] = h[:, :DH]
    o_ref[1] = h[:, DH:]


def _mid_body(hist_ref, s_ref, b1_ref, w2_ref, o_ref):
    dcol = _dinv_col(hist_ref)
    t = (jnp.concatenate([s_ref[0], s_ref[1]], axis=1).astype(jnp.float32)
         * dcol + b1_ref[...])
    a = jnp.maximum(t, 0.0)
    o_ref[...] = (jnp.dot(a, w2_ref[...], precision=lax.Precision.HIGHEST,
                          preferred_element_type=jnp.float32)
                  * dcol).astype(jnp.bfloat16)


def _fin_body(hist_ref, s_ref, g_ref, b2_ref, o_ref):
    dcol = _dinv_col(hist_ref)
    s = (s_ref[0].astype(jnp.float32) + s_ref[1].astype(jnp.float32)
         - g_ref[...].astype(jnp.float32))
    o_ref[...] = s * dcol + b2_ref[...]


def _row_spec(d):
    return pl.BlockSpec((2, RB, d), lambda i: (0, i, 0))


def _full_spec(shape):
    return pl.BlockSpec(shape, lambda i: tuple(0 for _ in shape))


def kernel(x, edge_index, W1, b1, W2, b2):
    ei = edge_index.astype(jnp.int32)
    pad = E_PAD - E
    src = jnp.concatenate([ei[0], jnp.zeros((pad,), jnp.int32)])
    pad_dst = N + jnp.arange(pad, dtype=jnp.int32) % (N_PAD - N)
    dst = jnp.concatenate([ei[1], pad_dst])
    src16, dst16 = src.reshape(NS, CHT, K), dst.reshape(NS, CHT, K)
    src32, dst32 = src.reshape(NW, CH, K), dst.reshape(NW, CH, K)
    xp = jnp.concatenate([x, jnp.zeros((N_PAD - N, D1), x.dtype)])
    const = jnp.concatenate([jnp.ones((K, DW), jnp.float32),
                             jnp.zeros((RT, DW), jnp.float32)])

    hist = _deg_kernel(dst32, const).reshape(NC, N_PAD, DW)

    g1 = pl.pallas_call(
        _mm1_body,
        grid=(GR,),
        out_shape=jax.ShapeDtypeStruct((NC, N_PAD, DH), jnp.bfloat16),
        in_specs=[_row_spec(DW),
                  pl.BlockSpec((RB, D1), lambda i: (i, 0)),
                  _full_spec((D1, D1))],
        out_specs=_row_spec(DH),
    )(hist, xp, W1)

    s1 = _agg1(g1, src16, dst16).reshape(NC, N_PAD, DH)

    g2 = pl.pallas_call(
        _mid_body,
        grid=(GR,),
        out_shape=jax.ShapeDtypeStruct((N_PAD, D2), jnp.bfloat16),
        in_specs=[_row_spec(DW),
                  _row_spec(DH),
                  _full_spec((1, D1)),
                  _full_spec((D1, D2))],
        out_specs=pl.BlockSpec((RB, D2), lambda i: (i, 0)),
    )(hist, s1, b1.reshape(1, D1), W2)

    s2 = _agg2(g2, src32, dst32).reshape(NC, N_PAD, D2)

    out = pl.pallas_call(
        _fin_body,
        grid=(GR,),
        out_shape=jax.ShapeDtypeStruct((N_PAD, D2), jnp.float32),
        in_specs=[_row_spec(DW),
                  _row_spec(D2),
                  pl.BlockSpec((RB, D2), lambda i: (i, 0)),
                  _full_spec((1, D2))],
        out_specs=pl.BlockSpec((RB, D2), lambda i: (i, 0)),
    )(hist, s2, g2, b2.reshape(1, D2))

    return out[:N]


# speedup vs baseline: 1.7461x; 1.0707x over previous
import functools

import jax
import jax.numpy as jnp
from jax import lax
from jax.experimental import pallas as pl
from jax.experimental.pallas import tpu as pltpu
from jax.experimental.pallas import tpu_sc as plsc

N = 10000
E = 320000
D1 = 128
DH = 64
D2 = 64
NC, NS = 2, 16
NW = NC * NS
K = 128
CH = 80
CHT = NC * CH
E_PAD = NW * CH * K
N_PAD = 10240
RT = N_PAD // NS
DW = 8
NBUF = 5
LOOK = 3

_sc_mesh = plsc.VectorSubcoreMesh(
    core_axis_name="c", subcore_axis_name="s", num_cores=NC, num_subcores=NS)
_sc_params = pltpu.CompilerParams(use_tc_tiling_on_sc=False)


@functools.partial(
    pl.kernel,
    out_type=jax.ShapeDtypeStruct((NC * N_PAD, DW), jnp.float32),
    mesh=_sc_mesh,
    scratch_types=[
        pltpu.VMEM((CH, K), jnp.int32),
        pltpu.VMEM((K + RT, DW), jnp.float32),
        pltpu.SemaphoreType.DMA,
        pltpu.VMEM_SHARED((N_PAD, DW), jnp.float32),
    ],
    compiler_params=_sc_params,
)
def _deg_kernel(dst_hbm, const_hbm, out_hbm, dstv, cv, dsem, acc):
    c = lax.axis_index("c")
    s = lax.axis_index("s")
    wid = c * NS + s
    pltpu.sync_copy(dst_hbm.at[wid], dstv)
    pltpu.sync_copy(const_hbm, cv)
    pltpu.sync_copy(cv.at[pl.ds(K, RT)], acc.at[pl.ds(s * RT, RT)])
    plsc.subcore_barrier()
    ones = cv.at[pl.ds(0, K)]
    FD = 8

    def grp(g, carry):
        base = g * FD
        for j in range(FD):
            pltpu.async_copy(ones, acc.at[dstv.at[base + j]], dsem, add=True)
        for j in range(FD):
            pltpu.make_async_copy(ones, acc.at[dstv.at[base + j]], dsem).wait()
        return carry

    lax.fori_loop(0, CH // FD, grp, 0)
    plsc.subcore_barrier()
    pltpu.sync_copy(acc.at[pl.ds(s * RT, RT)],
                    out_hbm.at[pl.ds(c * N_PAD + s * RT, RT)])


def _make_agg(D, split_cols, dtype=jnp.float32):
    chunks = CHT if split_cols else CH

    @functools.partial(
        pl.kernel,
        out_type=jax.ShapeDtypeStruct((NC * N_PAD, D), dtype),
        mesh=_sc_mesh,
        scratch_types=[
            pltpu.VMEM((chunks, K), jnp.int32),
            pltpu.VMEM((chunks, K), jnp.int32),
            [pltpu.VMEM((K, D), dtype) for _ in range(NBUF)],
            [pltpu.SemaphoreType.DMA for _ in range(NBUF)],
            [pltpu.SemaphoreType.DMA for _ in range(NBUF)],
            pltpu.VMEM_SHARED((N_PAD, D), dtype),
        ],
        compiler_params=_sc_params,
    )
    def _agg(g_hbm, src_hbm, dst_hbm, out_hbm, srcv, dstv, bufs, gsems, ssems,
             acc):
        c = lax.axis_index("c")
        s = lax.axis_index("s")
        if split_cols:
            gref = g_hbm.at[c]
            erow = s
        else:
            gref = g_hbm
            erow = c * NS + s
        rows = pl.ds(s * RT, RT)
        pltpu.sync_copy(gref.at[rows], acc.at[rows])
        pltpu.sync_copy(src_hbm.at[erow], srcv)
        pltpu.sync_copy(dst_hbm.at[erow], dstv)
        plsc.subcore_barrier()
        for j in range(LOOK):
            pltpu.async_copy(gref.at[srcv.at[j]], bufs[j], gsems[j])

        def grp(g, carry):
            base = g * NBUF
            for b in range(NBUF):
                ch = base + b
                pltpu.make_async_copy(
                    gref.at[srcv.at[ch]], bufs[b], gsems[b]).wait()
                pltpu.async_copy(bufs[b], acc.at[dstv.at[ch]], ssems[b],
                                 add=True)
                f = ch + LOOK
                bf = (b + LOOK) % NBUF

                @pl.when(f < chunks)
                def _issue(f=f, bf=bf):
                    @pl.when(f >= NBUF)
                    def _drain():
                        pltpu.make_async_copy(
                            bufs[bf], acc.at[dstv.at[f - NBUF]],
                            ssems[bf]).wait()

                    pltpu.async_copy(gref.at[srcv.at[f]], bufs[bf], gsems[bf])
            return carry

        lax.fori_loop(0, chunks // NBUF, grp, 0)
        for b in range(NBUF):
            ch = chunks - NBUF + b
            pltpu.make_async_copy(bufs[b], acc.at[dstv.at[ch]],
                                  ssems[b]).wait()
        plsc.subcore_barrier()
        pltpu.sync_copy(acc.at[pl.ds(s * RT, RT)],
                        out_hbm.at[pl.ds(c * N_PAD + s * RT, RT)])

    return _agg


_agg1 = _make_agg(DH, split_cols=True, dtype=jnp.bfloat16)
_agg2 = _make_agg(D2 // 2, split_cols=True, dtype=jnp.bfloat16)

RB = 1024
GR = N_PAD // RB


def _dinv_col(hist_ref):
    deg = hist_ref[0] + hist_ref[1] + 1.0
    return lax.rsqrt(deg)[:, 0:1]


def _mm1_body(hist_ref, x_ref, w_ref, o_ref):
    h = (jnp.dot(x_ref[...], w_ref[...], precision=lax.Precision.HIGHEST,
                 preferred_element_type=jnp.float32)
         * _dinv_col(hist_ref)).astype(jnp.bfloat16)
    o_ref[0] = h[:, :DH]
    o_ref[1] = h[:, DH:]


def _mid_body(hist_ref, s_ref, b1_ref, w2_ref, o_ref):
    dcol = _dinv_col(hist_ref)
    t = (jnp.concatenate([s_ref[0], s_ref[1]], axis=1).astype(jnp.float32)
         * dcol + b1_ref[...])
    a = jnp.maximum(t, 0.0)
    g = (jnp.dot(a, w2_ref[...], precision=lax.Precision.HIGHEST,
                 preferred_element_type=jnp.float32)
         * dcol).astype(jnp.bfloat16)
    o_ref[0] = g[:, :D2 // 2]
    o_ref[1] = g[:, D2 // 2:]


def _fin_body(hist_ref, s_ref, b2_ref, o_ref):
    dcol = _dinv_col(hist_ref)
    s = jnp.concatenate([s_ref[0], s_ref[1]], axis=1).astype(jnp.float32)
    o_ref[...] = s * dcol + b2_ref[...]


def _row_spec(d):
    return pl.BlockSpec((2, RB, d), lambda i: (0, i, 0))


def _full_spec(shape):
    return pl.BlockSpec(shape, lambda i: tuple(0 for _ in shape))


def kernel(x, edge_index, W1, b1, W2, b2):
    ei = edge_index.astype(jnp.int32)
    pad = E_PAD - E
    src = jnp.concatenate([ei[0], jnp.zeros((pad,), jnp.int32)])
    pad_dst = N + jnp.arange(pad, dtype=jnp.int32) % (N_PAD - N)
    dst = jnp.concatenate([ei[1], pad_dst])
    src16, dst16 = src.reshape(NS, CHT, K), dst.reshape(NS, CHT, K)
    src32, dst32 = src.reshape(NW, CH, K), dst.reshape(NW, CH, K)
    xp = jnp.concatenate([x, jnp.zeros((N_PAD - N, D1), x.dtype)])
    const = jnp.concatenate([jnp.ones((K, DW), jnp.float32),
                             jnp.zeros((RT, DW), jnp.float32)])

    hist = _deg_kernel(dst32, const).reshape(NC, N_PAD, DW)

    g1 = pl.pallas_call(
        _mm1_body,
        grid=(GR,),
        out_shape=jax.ShapeDtypeStruct((NC, N_PAD, DH), jnp.bfloat16),
        in_specs=[_row_spec(DW),
                  pl.BlockSpec((RB, D1), lambda i: (i, 0)),
                  _full_spec((D1, D1))],
        out_specs=_row_spec(DH),
    )(hist, xp, W1)

    s1 = _agg1(g1, src16, dst16).reshape(NC, N_PAD, DH)

    g2 = pl.pallas_call(
        _mid_body,
        grid=(GR,),
        out_shape=jax.ShapeDtypeStruct((NC, N_PAD, D2 // 2), jnp.bfloat16),
        in_specs=[_row_spec(DW),
                  _row_spec(DH),
                  _full_spec((1, D1)),
                  _full_spec((D1, D2))],
        out_specs=_row_spec(D2 // 2),
    )(hist, s1, b1.reshape(1, D1), W2)

    s2 = _agg2(g2, src16, dst16).reshape(NC, N_PAD, D2 // 2)

    out = pl.pallas_call(
        _fin_body,
        grid=(GR,),
        out_shape=jax.ShapeDtypeStruct((N_PAD, D2), jnp.float32),
        in_specs=[_row_spec(DW),
                  _row_spec(D2 // 2),
                  _full_spec((1, D2))],
        out_specs=pl.BlockSpec((RB, D2), lambda i: (i, 0)),
    )(hist, s2, b2.reshape(1, D2))

    return out[:N]
